# Initial kernel scaffold; baseline (speedup 1.0000x reference)
import functools

import jax
import jax.numpy as jnp
from jax import lax
from jax.experimental import pallas as pl
from jax.experimental.pallas import tpu as pltpu
from jax.experimental.pallas import tpu_sc as plsc

NC = 2
NS = 16
L = 16
NW = NC * NS


def _relmm_body(feat_ref, w_ref, y_ref):
    y_ref[0] = jnp.dot(feat_ref[...], w_ref[0],
                       preferred_element_type=jnp.float32)


def _rel_matmul(feat, weight, bn):
    n, d = feat.shape
    r = weight.shape[0]
    return pl.pallas_call(
        _relmm_body,
        grid=(n // bn, r),
        in_specs=[
            pl.BlockSpec((bn, d), lambda nb, rb: (nb, 0)),
            pl.BlockSpec((1, d, d), lambda nb, rb: (rb, 0, 0)),
        ],
        out_specs=pl.BlockSpec((1, bn, d), lambda nb, rb: (rb, nb, 0)),
        out_shape=jax.ShapeDtypeStruct((r, n, d), jnp.float32),
    )(feat, weight)


def _add_body(a_ref, b_ref, o_ref):
    o_ref[...] = a_ref[...] + b_ref[...]


def _add_halves(partials, bn):
    two_n, d = partials.shape
    n = two_n // 2
    nb = n // bn
    return pl.pallas_call(
        _add_body,
        grid=(nb,),
        in_specs=[
            pl.BlockSpec((bn, d), lambda i: (i, 0)),
            pl.BlockSpec((bn, d), lambda i, _nb=nb: (i + _nb, 0)),
        ],
        out_specs=pl.BlockSpec((bn, d), lambda i: (i, 0)),
        out_shape=jax.ShapeDtypeStruct((n, d), jnp.float32),
    )(partials, partials)


def _make_sc_gather_scatter(n, d, e, c):
    ew = e // NW
    nchunk = ew // c
    rt = n // NS

    mesh = plsc.VectorSubcoreMesh(core_axis_name="c", subcore_axis_name="s",
                                  num_cores=NC, num_subcores=NS)

    @functools.partial(
        pl.kernel,
        out_type=jax.ShapeDtypeStruct((2 * n, d), jnp.float32),
        mesh=mesh,
        scratch_types=[
            pltpu.VMEM((c,), jnp.int32),
            pltpu.VMEM((c,), jnp.int32),
            pltpu.VMEM((c,), jnp.int32),
            pltpu.VMEM((c,), jnp.int32),
            pltpu.VMEM((c, d), jnp.float32),
            pltpu.VMEM_SHARED((n, d), jnp.float32),
            pltpu.SemaphoreType.DMA,
        ],
    )
    def sc_kernel(y_hbm, et_hbm, src_hbm, dst_hbm, zeros_hbm, out_hbm,
                  et_v, src_v, dst_v, key_v, rows_v, acc, sem):
        ci = lax.axis_index("c")
        si = lax.axis_index("s")
        wid = ci * NS + si
        base = wid * ew

        pltpu.sync_copy(zeros_hbm, acc.at[pl.ds(si * rt, rt)])
        plsc.subcore_barrier()

        def chunk_body(i, _):
            off = base + i * c
            pltpu.sync_copy(et_hbm.at[pl.ds(off, c)], et_v)
            pltpu.sync_copy(src_hbm.at[pl.ds(off, c)], src_v)
            pltpu.sync_copy(dst_hbm.at[pl.ds(off, c)], dst_v)
            for j in range(c // L):
                sl = pl.ds(j * L, L)
                key_v[sl] = et_v[sl] * n + src_v[sl]
            pltpu.async_copy(y_hbm.at[key_v], rows_v, sem).wait()
            pltpu.sync_copy(rows_v, acc.at[dst_v], add=True)
            return 0

        lax.fori_loop(0, nchunk, chunk_body, 0)

        plsc.subcore_barrier()
        pltpu.sync_copy(acc.at[pl.ds(si * rt, rt)],
                        out_hbm.at[pl.ds(ci * n + si * rt, rt)])

    return sc_kernel


def kernel(feat, edge_index, etypes, E_per_rel, weight):
    n, d = feat.shape
    r = weight.shape[0]
    e = etypes.shape[0]

    y = _rel_matmul(feat, weight, bn=1000).reshape(r * n, d)

    src = edge_index[0]
    dst = edge_index[1]
    zeros = jnp.zeros((n // NS, d), jnp.float32)

    sc = _make_sc_gather_scatter(n, d, e, c=80)
    partials = sc(y, etypes, src, dst, zeros)

    return _add_halves(partials, bn=1000)


# speedup vs baseline: 12.6291x; 12.6291x over previous
import functools

import jax
import jax.numpy as jnp
from jax import lax
from jax.experimental import pallas as pl
from jax.experimental.pallas import tpu as pltpu
from jax.experimental.pallas import tpu_sc as plsc

NC = 2
NS = 16
L = 16
NW = NC * NS


def _relmm_body(feat_ref, w_ref, y_ref):
    y_ref[0] = jnp.dot(feat_ref[...], w_ref[0],
                       preferred_element_type=jnp.float32)


def _rel_matmul(feat, weight, bn):
    n, d = feat.shape
    r = weight.shape[0]
    return pl.pallas_call(
        _relmm_body,
        grid=(n // bn, r),
        in_specs=[
            pl.BlockSpec((bn, d), lambda nb, rb: (nb, 0)),
            pl.BlockSpec((1, d, d), lambda nb, rb: (rb, 0, 0)),
        ],
        out_specs=pl.BlockSpec((1, bn, d), lambda nb, rb: (rb, nb, 0)),
        out_shape=jax.ShapeDtypeStruct((r, n, d), jnp.float32),
    )(feat, weight)


def _add_body(a_ref, b_ref, o_ref):
    o_ref[...] = a_ref[...] + b_ref[...]


def _add_halves(a, b, bn):
    n, d = a.shape
    return pl.pallas_call(
        _add_body,
        grid=(n // bn,),
        in_specs=[
            pl.BlockSpec((bn, d), lambda i: (i, 0)),
            pl.BlockSpec((bn, d), lambda i: (i, 0)),
        ],
        out_specs=pl.BlockSpec((bn, d), lambda i: (i, 0)),
        out_shape=jax.ShapeDtypeStruct((n, d), jnp.float32),
    )(a, b)


def _make_sc_gather_scatter(n, d, e, c):
    ew = e // NW
    nchunk = ew // c
    rt = ((-(-n // NS) + 7) // 8) * 8
    npad = rt * NS

    mesh = plsc.VectorSubcoreMesh(core_axis_name="c", subcore_axis_name="s",
                                  num_cores=NC, num_subcores=NS)

    @functools.partial(
        pl.kernel,
        out_type=jax.ShapeDtypeStruct((2, npad, d), jnp.float32),
        mesh=mesh,
        scratch_types=[
            pltpu.VMEM((c,), jnp.int32),
            pltpu.VMEM((c,), jnp.int32),
            pltpu.VMEM((c,), jnp.int32),
            pltpu.VMEM((c,), jnp.int32),
            pltpu.VMEM((c, d), jnp.float32),
            pltpu.VMEM_SHARED((npad, d), jnp.float32),
            pltpu.SemaphoreType.DMA,
        ],
    )
    def sc_kernel(y_hbm, et_hbm, src_hbm, dst_hbm, zeros_hbm, out_hbm,
                  et_v, src_v, dst_v, key_v, rows_v, acc, sem):
        ci = lax.axis_index("c")
        si = lax.axis_index("s")
        wid = ci * NS + si
        base = wid * ew

        pltpu.sync_copy(zeros_hbm, acc.at[pl.ds(si * rt, rt)])
        plsc.subcore_barrier()

        def chunk_body(i, _):
            off = base + i * c
            pltpu.sync_copy(et_hbm.at[pl.ds(off, c)], et_v)
            pltpu.sync_copy(src_hbm.at[pl.ds(off, c)], src_v)
            pltpu.sync_copy(dst_hbm.at[pl.ds(off, c)], dst_v)
            for j in range(c // L):
                sl = pl.ds(j * L, L)
                key_v[sl] = et_v[sl] * n + src_v[sl]
            pltpu.async_copy(y_hbm.at[key_v], rows_v, sem).wait()
            pltpu.sync_copy(rows_v, acc.at[dst_v], add=True)
            return 0

        lax.fori_loop(0, nchunk, chunk_body, 0)

        plsc.subcore_barrier()
        pltpu.sync_copy(acc.at[pl.ds(si * rt, rt)],
                        out_hbm.at[ci, pl.ds(si * rt, rt)])

    return sc_kernel


def kernel(feat, edge_index, etypes, E_per_rel, weight):
    n, d = feat.shape
    r = weight.shape[0]
    e = etypes.shape[0]

    y = _rel_matmul(feat, weight, bn=1000).reshape(r * n, d)

    src = edge_index[0]
    dst = edge_index[1]
    rt = ((-(-n // NS) + 7) // 8) * 8
    zeros = jnp.zeros((rt, d), jnp.float32)

    sc = _make_sc_gather_scatter(n, d, e, c=80)
    partials = sc(y, etypes, src, dst, zeros)

    return _add_halves(partials[0, :n], partials[1, :n], bn=1000)
